# baseline (speedup 1.0000x reference)
import functools
import jax
import jax.numpy as jnp
from jax import lax
from jax.experimental import pallas as pl
from jax.experimental.pallas import tpu as pltpu
from jax.experimental.pallas import tpu_sc as plsc

N_NODES = 2048
N_EDGES = 65536
N_TOKENS = 4096
IN_DIM = 128
HID = 64
HEADS = 4
OUT_DIM = 128
BB = 16
NODE_COUNT = 128
NUM_CLASSES = 16
LSTM_H = 100

NC = 2
NS = 16
NW = NC * NS


def _prep1_body(x_ref, w1_ref, al_ref, ar_ref, tok_ref, wg1_ref,
                h_ref, el_ref, er_ref, p_ref):
    h = jnp.dot(x_ref[...], w1_ref[...], preferred_element_type=jnp.float32)
    h_ref[...] = h
    row = lax.broadcasted_iota(jnp.int32, (HEADS * HID, HEADS), 0)
    col = lax.broadcasted_iota(jnp.int32, (HEADS * HID, HEADS), 1)
    sel = (row // HID) == col
    amat = jnp.where(sel, al_ref[...].reshape(HEADS * HID, 1), 0.0)
    bmat = jnp.where(sel, ar_ref[...].reshape(HEADS * HID, 1), 0.0)
    el_ref[...] = jnp.dot(h, amat, preferred_element_type=jnp.float32)
    er_ref[...] = jnp.dot(h, bmat, preferred_element_type=jnp.float32)
    p_ref[...] = jnp.dot(tok_ref[...], wg1_ref[...],
                         preferred_element_type=jnp.float32)


def _prep1(x, w1, al1f, ar1f, tok, wg1):
    return pl.pallas_call(
        _prep1_body,
        out_shape=(
            jax.ShapeDtypeStruct((N_NODES, HEADS * HID), jnp.float32),
            jax.ShapeDtypeStruct((N_NODES, HEADS), jnp.float32),
            jax.ShapeDtypeStruct((N_NODES, HEADS), jnp.float32),
            jax.ShapeDtypeStruct((N_TOKENS, HID), jnp.float32),
        ),
    )(x, w1, al1f, ar1f, tok, wg1)


_CAP = 4096
_SCAN_BLK = 4096
_ROWS = N_NODES // NW


def _make_gat_scatter_sc(n_heads, pass_rows, with_scan):
    n_pass = _ROWS // pass_rows
    pn = pass_rows * N_NODES
    nh_pn = n_heads * pn

    mesh = plsc.VectorSubcoreMesh(core_axis_name="c", subcore_axis_name="s",
                                  num_cores=NC, num_subcores=NS)

    out_type = [
        jax.ShapeDtypeStruct((n_heads * N_NODES * N_NODES,), jnp.float32),
    ]
    scratch = [
        pltpu.VMEM((N_NODES * n_heads,), jnp.float32),
        pltpu.VMEM((N_NODES * n_heads,), jnp.float32),
        pltpu.VMEM((_CAP,), jnp.int32),
        pltpu.VMEM((_CAP,), jnp.int32),
        pltpu.VMEM((16,), jnp.int32),
        pltpu.VMEM((n_heads * _CAP,), jnp.float32),
        pltpu.VMEM((2 * nh_pn,), jnp.float32),
        pltpu.SemaphoreType.DMA((2,)),
    ]
    if with_scan:
        out_type += [
            jax.ShapeDtypeStruct((NW * _CAP,), jnp.int32),
            jax.ShapeDtypeStruct((NW * _CAP,), jnp.int32),
            jax.ShapeDtypeStruct((NW * 16,), jnp.int32),
        ]
        scratch += [
            pltpu.VMEM((2 * _SCAN_BLK,), jnp.int32),
            pltpu.VMEM((2 * _SCAN_BLK,), jnp.int32),
            pltpu.SemaphoreType.DMA((2,)),
            pltpu.SemaphoreType.DMA((2,)),
        ]

    @functools.partial(
        pl.kernel,
        out_type=tuple(out_type),
        mesh=mesh,
        compiler_params=pltpu.CompilerParams(needs_layout_passes=False),
        scratch_types=scratch,
    )
    def gat_kernel(el_hbm, er_hbm, sa_hbm, da_hbm, *rest):
        if with_scan:
            (a_hbm, ssrc_hbm, sdst_hbm, cnt_hbm,
             el_v, er_v, sel_src, sel_dst, cbuf, w_v, a_v, sem_a,
             blk_src, blk_dst, sem_s, sem_d) = rest
        else:
            (cn_hbm, a_hbm,
             el_v, er_v, sel_src, sel_dst, cbuf, w_v, a_v, sem_a) = rest
        cid = lax.axis_index("c")
        sid = lax.axis_index("s")
        wid = cid * NS + sid
        iota16 = lax.iota(jnp.int32, 16)
        zero16 = jnp.zeros((16,), jnp.float32)

        pltpu.sync_copy(el_hbm, el_v)
        pltpu.sync_copy(er_hbm, er_v)

        if with_scan:
            def zsel(i, carry):
                sel_src[pl.ds(i * 16, 16)] = jnp.zeros((16,), jnp.int32)
                sel_dst[pl.ds(i * 16, 16)] = jnp.zeros((16,), jnp.int32)
                return carry
            lax.fori_loop(0, _CAP // 16, zsel, 0)
            n_blk = N_EDGES // _SCAN_BLK

            def issue_blk(b):
                par = (b % 2) * _SCAN_BLK
                pltpu.async_copy(
                    sa_hbm.at[pl.ds(b * _SCAN_BLK, _SCAN_BLK)],
                    blk_src.at[pl.ds(par, _SCAN_BLK)], sem_s.at[b % 2])
                pltpu.async_copy(
                    da_hbm.at[pl.ds(b * _SCAN_BLK, _SCAN_BLK)],
                    blk_dst.at[pl.ds(par, _SCAN_BLK)], sem_d.at[b % 2])

            issue_blk(0)
            cnt = jnp.int32(0)
            for blk in range(n_blk):
                if blk + 1 < n_blk:
                    issue_blk(blk + 1)
                par = (blk % 2) * _SCAN_BLK
                pltpu.make_async_copy(
                    sa_hbm.at[pl.ds(blk * _SCAN_BLK, _SCAN_BLK)],
                    blk_src.at[pl.ds(par, _SCAN_BLK)],
                    sem_s.at[blk % 2]).wait()
                pltpu.make_async_copy(
                    da_hbm.at[pl.ds(blk * _SCAN_BLK, _SCAN_BLK)],
                    blk_dst.at[pl.ds(par, _SCAN_BLK)],
                    sem_d.at[blk % 2]).wait()

                def scan_body(ci, off, par=par):
                    sv = blk_src[pl.ds(par + ci * 16, 16)]
                    dv = blk_dst[pl.ds(par + ci * 16, 16)]
                    m = (dv >> 6) == wid
                    plsc.store_compressed(sel_src.at[pl.ds(off, 16)], sv,
                                          mask=m)
                    plsc.store_compressed(sel_dst.at[pl.ds(off, 16)], dv,
                                          mask=m)
                    nsel = plsc.all_reduce_population_count(m)
                    return off + nsel[0]

                cnt = lax.fori_loop(0, _SCAN_BLK // 16, scan_body, cnt)
            cbuf[...] = jnp.full((16,), cnt, jnp.int32)
            pltpu.sync_copy(cbuf, cnt_hbm.at[pl.ds(wid * 16, 16)])
            pltpu.sync_copy(sel_src, ssrc_hbm.at[pl.ds(wid * _CAP, _CAP)])
            pltpu.sync_copy(sel_dst, sdst_hbm.at[pl.ds(wid * _CAP, _CAP)])
        else:
            pltpu.sync_copy(sa_hbm.at[pl.ds(wid * _CAP, _CAP)], sel_src)
            pltpu.sync_copy(da_hbm.at[pl.ds(wid * _CAP, _CAP)], sel_dst)
            pltpu.sync_copy(cn_hbm.at[pl.ds(wid * 16, 16)], cbuf)
            cnt = cbuf[pl.ds(0, 16)][0]

        nc_chunks = (cnt + 15) >> 4

        def wpass(ci, carry):
            sv = sel_src[pl.ds(ci * 16, 16)] & (N_NODES - 1)
            dv = sel_dst[pl.ds(ci * 16, 16)] & (N_NODES - 1)
            mv = (ci * 16 + iota16) < cnt
            for h in range(n_heads):
                elh = plsc.load_gather(el_v, [sv * n_heads + h])
                erh = plsc.load_gather(er_v, [dv * n_heads + h])
                e = elh + erh
                e = jnp.where(e >= 0.0, e, 0.2 * e)
                w = jnp.where(mv, jnp.exp(e), 0.0)
                w_v[pl.ds(h * _CAP + ci * 16, 16)] = w
            return carry

        lax.fori_loop(0, nc_chunks, wpass, 0)

        def wb_copy(q, h):
            par = (q % 2) * nh_pn
            off = (h * N_NODES + wid * _ROWS + q * pass_rows) * N_NODES
            return pltpu.make_async_copy(a_v.at[pl.ds(par + h * pn, pn)],
                                         a_hbm.at[pl.ds(off, pn)],
                                         sem_a.at[q % 2])

        for q in range(n_pass):
            par = (q % 2) * nh_pn
            if q >= 2:
                for h in range(n_heads):
                    wb_copy(q - 2, h).wait()

            def zloop(i, carry, par=par):
                for u in range(8):
                    a_v[pl.ds(par + (i * 8 + u) * 16, 16)] = zero16
                return carry
            lax.fori_loop(0, nh_pn // 128, zloop, 0)

            def spass(ci, carry, q=q, par=par):
                sv = sel_src[pl.ds(ci * 16, 16)] & (N_NODES - 1)
                dv = sel_dst[pl.ds(ci * 16, 16)]
                if n_pass > 1:
                    pr_shift = pass_rows.bit_length() - 1
                    mq = ((dv >> pr_shift) & (n_pass - 1)) == q
                else:
                    mq = None
                idx = (dv & (pass_rows - 1)) * N_NODES + sv + par
                for h in range(n_heads):
                    w = w_v[pl.ds(h * _CAP + ci * 16, 16)]
                    plsc.addupdate_scatter(a_v, [idx + h * pn], w, mask=mq)
                return carry

            lax.fori_loop(0, nc_chunks, spass, 0)

            for h in range(n_heads):
                par2 = (q % 2) * nh_pn
                off = (h * N_NODES + wid * _ROWS + q * pass_rows) * N_NODES
                pltpu.async_copy(a_v.at[pl.ds(par2 + h * pn, pn)],
                                 a_hbm.at[pl.ds(off, pn)],
                                 sem_a.at[q % 2])

        for q in range(max(0, n_pass - 2), n_pass):
            for h in range(n_heads):
                wb_copy(q, h).wait()

    return gat_kernel


_gat_sc_1 = _make_gat_scatter_sc(HEADS, 4, True)
_gat_sc_2 = _make_gat_scatter_sc(1, 16, False)


_BLK = 256


def _prep2_body(a_ref, h_ref, w2_ref, al_ref, ar_ref,
                h2_ref, el_ref, er_ref):
    parts = []
    for h in range(HEADS):
        a = a_ref[h]
        num = jnp.dot(a, h_ref[:, h * HID:(h + 1) * HID],
                      preferred_element_type=jnp.float32)
        den = jnp.sum(a, axis=1, keepdims=True) + 1e-9
        parts.append(num / den)
    h1 = jnp.maximum(jnp.concatenate(parts, axis=1), 0.0)
    h2 = jnp.dot(h1, w2_ref[...], preferred_element_type=jnp.float32)
    h2_ref[...] = h2
    el_ref[...] = jnp.dot(h2, al_ref[...], preferred_element_type=jnp.float32)
    er_ref[...] = jnp.dot(h2, ar_ref[...], preferred_element_type=jnp.float32)


def _prep2(a1, h1pre, w2, al2f, ar2f):
    nb = N_NODES // _BLK
    return pl.pallas_call(
        _prep2_body,
        grid=(nb,),
        in_specs=[
            pl.BlockSpec((HEADS, _BLK, N_NODES), lambda i: (0, i, 0)),
            pl.BlockSpec((N_NODES, HEADS * HID), lambda i: (0, 0)),
            pl.BlockSpec((HEADS * HID, OUT_DIM), lambda i: (0, 0)),
            pl.BlockSpec((OUT_DIM, 1), lambda i: (0, 0)),
            pl.BlockSpec((OUT_DIM, 1), lambda i: (0, 0)),
        ],
        out_specs=(
            pl.BlockSpec((_BLK, OUT_DIM), lambda i: (i, 0)),
            pl.BlockSpec((_BLK, 1), lambda i: (i, 0)),
            pl.BlockSpec((_BLK, 1), lambda i: (i, 0)),
        ),
        out_shape=(
            jax.ShapeDtypeStruct((N_NODES, OUT_DIM), jnp.float32),
            jax.ShapeDtypeStruct((N_NODES, 1), jnp.float32),
            jax.ShapeDtypeStruct((N_NODES, 1), jnp.float32),
        ),
    )(a1, h1pre, w2, al2f, ar2f)


_GCN_BLK = 512


def _gcn_b_body(adj_ref, p_ref, wg2_ref, y_ref):
    acc = jnp.dot(adj_ref[...], p_ref[...], preferred_element_type=jnp.float32)
    y_ref[...] = jnp.dot(jnp.maximum(acc, 0.0), wg2_ref[...],
                         preferred_element_type=jnp.float32)


def _gcn_b(adj, p, wg2):
    nb = N_TOKENS // _GCN_BLK
    return pl.pallas_call(
        _gcn_b_body,
        grid=(nb,),
        in_specs=[
            pl.BlockSpec((_GCN_BLK, N_TOKENS), lambda i: (i, 0)),
            pl.BlockSpec((N_TOKENS, HID), lambda i: (0, 0)),
            pl.BlockSpec((HID, OUT_DIM), lambda i: (0, 0)),
        ],
        out_specs=pl.BlockSpec((_GCN_BLK, OUT_DIM), lambda i: (i, 0)),
        out_shape=jax.ShapeDtypeStruct((N_TOKENS, OUT_DIM), jnp.float32),
    )(adj, p, wg2)


def _gcn_c_body(adj_ref, y_ref, a2_ref, h2p_ref, t_ref, h2_ref):
    t_ref[...] = jnp.dot(adj_ref[...], y_ref[...],
                         preferred_element_type=jnp.float32)
    a = a2_ref[...]
    num = jnp.dot(a, h2p_ref[...], preferred_element_type=jnp.float32)
    den = jnp.sum(a, axis=1, keepdims=True) + 1e-9
    h2_ref[...] = num / den


def _gcn_c(adj, y, a2, h2pre):
    nb = N_TOKENS // _GCN_BLK
    hblk = N_NODES // nb
    return pl.pallas_call(
        _gcn_c_body,
        grid=(nb,),
        in_specs=[
            pl.BlockSpec((_GCN_BLK, N_TOKENS), lambda i: (i, 0)),
            pl.BlockSpec((N_TOKENS, OUT_DIM), lambda i: (0, 0)),
            pl.BlockSpec((hblk, N_NODES), lambda i: (i, 0)),
            pl.BlockSpec((N_NODES, OUT_DIM), lambda i: (0, 0)),
        ],
        out_specs=(
            pl.BlockSpec((_GCN_BLK, OUT_DIM), lambda i: (i, 0)),
            pl.BlockSpec((hblk, OUT_DIM), lambda i: (i, 0)),
        ),
        out_shape=(
            jax.ShapeDtypeStruct((N_TOKENS, OUT_DIM), jnp.float32),
            jax.ShapeDtypeStruct((N_NODES, OUT_DIM), jnp.float32),
        ),
    )(adj, y, a2, h2pre)


def _make_gather_sc():
    mesh = plsc.VectorSubcoreMesh(core_axis_name="c", subcore_axis_name="s",
                                  num_cores=NC, num_subcores=NS)
    t_per_tile = NODE_COUNT // NW

    @functools.partial(
        pl.kernel,
        out_type=(
            jax.ShapeDtypeStruct((BB * NODE_COUNT, OUT_DIM), jnp.float32),
            jax.ShapeDtypeStruct((BB * NODE_COUNT, OUT_DIM), jnp.float32),
        ),
        mesh=mesh,
        compiler_params=pltpu.CompilerParams(needs_layout_passes=False),
        scratch_types=[
            pltpu.VMEM((BB * NODE_COUNT,), jnp.int32),
            pltpu.VMEM((BB * NODE_COUNT,), jnp.int32),
            pltpu.VMEM((16, OUT_DIM), jnp.float32),
            pltpu.VMEM((16, OUT_DIM), jnp.float32),
            pltpu.SemaphoreType.DMA,
            pltpu.SemaphoreType.DMA,
        ],
    )
    def gather_kernel(t_hbm, h2_hbm, lids_hbm, gids_hbm,
                      tok_out, inst_out,
                      lids_v, gids_v, rows_a, rows_b, sem_a, sem_b):
        cid = lax.axis_index("c")
        sid = lax.axis_index("s")
        wid = sid * NC + cid
        iota16 = lax.iota(jnp.int32, 16)
        pltpu.sync_copy(lids_hbm, lids_v)
        pltpu.sync_copy(gids_hbm, gids_v)

        def body(i, carry):
            t = wid * t_per_tile + i
            flat = iota16 * NODE_COUNT + t
            gv = plsc.load_gather(gids_v, [flat])
            ca = pltpu.async_copy(t_hbm.at[gv], rows_a, sem_a)
            lv = plsc.load_gather(lids_v, [flat])
            inst_idx = iota16 * NODE_COUNT + lv
            cb = pltpu.async_copy(h2_hbm.at[inst_idx], rows_b, sem_b)
            ca.wait()
            pltpu.sync_copy(rows_a, tok_out.at[pl.ds(t * 16, 16)])
            cb.wait()
            pltpu.sync_copy(rows_b, inst_out.at[pl.ds(t * 16, 16)])
            return carry

        lax.fori_loop(0, t_per_tile, body, 0)

    return gather_kernel


_gather_sc = _make_gather_sc()


def _lstm_body(tok_ref, inst_ref,
               wih0f_ref, whh0f_ref, b0f_ref,
               wih0b_ref, whh0b_ref, b0b_ref,
               wih1f_ref, whh1f_ref, b1f_ref,
               wih1b_ref, whh1b_ref, b1b_ref,
               wfc_ref, bfc_ref,
               out_ref,
               xga_ref, xgb_ref, xgcat_ref, l0_ref):
    cdims = (((1,), (1,)), ((), ()))
    T = NODE_COUNT
    H = LSTM_H
    z100 = jnp.zeros((H, H), jnp.float32)

    def whh_cat(wf, wb):
        blocks = []
        for g in range(4):
            blocks.append(jnp.concatenate([wf[g * H:(g + 1) * H], z100], 1))
            blocks.append(jnp.concatenate([z100, wb[g * H:(g + 1) * H]], 1))
        return jnp.concatenate(blocks, 0)

    def interleave(k, carry):
        kb = T - 1 - k
        xf = xga_ref[pl.ds(k * BB, BB), :]
        xb = xgb_ref[pl.ds(kb * BB, BB), :]
        parts = []
        for g in range(4):
            parts.append(xf[:, g * H:(g + 1) * H])
            parts.append(xb[:, g * H:(g + 1) * H])
        xgcat_ref[pl.ds(k * BB, BB), :] = jnp.concatenate(parts, 1)
        return carry

    def packed_cell(g, ccat):
        i = jax.nn.sigmoid(g[:, :2 * H])
        f = jax.nn.sigmoid(g[:, 2 * H:4 * H])
        gg = jnp.tanh(g[:, 4 * H:6 * H])
        o = jax.nn.sigmoid(g[:, 6 * H:])
        c2 = f * ccat + i * gg
        h2 = o * jnp.tanh(c2)
        return h2, c2

    tok = tok_ref[...]
    inst = inst_ref[...]

    def xgates0(w_ref, b_ref):
        w = w_ref[...]
        g = lax.dot_general(tok, w[:, :OUT_DIM], cdims,
                            preferred_element_type=jnp.float32)
        g = g + lax.dot_general(inst, w[:, OUT_DIM:], cdims,
                                preferred_element_type=jnp.float32)
        return g + b_ref[...]

    xga_ref[...] = xgates0(wih0f_ref, b0f_ref)
    xgb_ref[...] = xgates0(wih0b_ref, b0b_ref)
    lax.fori_loop(0, T, interleave, 0)
    w0 = whh_cat(whh0f_ref[...], whh0b_ref[...])

    def step0(k, carry):
        hcat, ccat = carry
        g = xgcat_ref[pl.ds(k * BB, BB), :] + lax.dot_general(
            hcat, w0, (((1,), (1,)), ((), ())),
            preferred_element_type=jnp.float32)
        hcat, ccat = packed_cell(g, ccat)
        kb = T - 1 - k
        l0_ref[pl.ds(k * BB, BB), :H] = hcat[:, :H]
        l0_ref[pl.ds(kb * BB, BB), H:] = hcat[:, H:]
        return hcat, ccat

    z = jnp.zeros((BB, 2 * H), jnp.float32)
    lax.fori_loop(0, T, step0, (z, z))

    l0 = l0_ref[...]
    xga_ref[...] = lax.dot_general(l0, wih1f_ref[...], cdims,
                                   preferred_element_type=jnp.float32) + b1f_ref[...]
    xgb_ref[...] = lax.dot_general(l0, wih1b_ref[...], cdims,
                                   preferred_element_type=jnp.float32) + b1b_ref[...]
    lax.fori_loop(0, T, interleave, 0)
    w1 = whh_cat(whh1f_ref[...], whh1b_ref[...])

    def step1(k, carry):
        hcat, ccat = carry
        g = xgcat_ref[pl.ds(k * BB, BB), :] + lax.dot_general(
            hcat, w1, (((1,), (1,)), ((), ())),
            preferred_element_type=jnp.float32)
        return packed_cell(g, ccat)

    hcat, _ = lax.fori_loop(0, T, step1, (z, z))
    out_ref[...] = jnp.dot(hcat, wfc_ref[...],
                           preferred_element_type=jnp.float32) + bfc_ref[...]


def _lstm(tok_tm, inst_tm, p):
    D1 = 2 * LSTM_H
    n = BB * NODE_COUNT
    b0f = (p['bih0f'] + p['bhh0f']).reshape(1, 4 * LSTM_H)
    b0b = (p['bih0b'] + p['bhh0b']).reshape(1, 4 * LSTM_H)
    b1f = (p['bih1f'] + p['bhh1f']).reshape(1, 4 * LSTM_H)
    b1b = (p['bih1b'] + p['bhh1b']).reshape(1, 4 * LSTM_H)
    return pl.pallas_call(
        _lstm_body,
        out_shape=jax.ShapeDtypeStruct((BB, NUM_CLASSES), jnp.float32),
        scratch_shapes=[
            pltpu.VMEM((n, 4 * LSTM_H), jnp.float32),
            pltpu.VMEM((n, 4 * LSTM_H), jnp.float32),
            pltpu.VMEM((n, 8 * LSTM_H), jnp.float32),
            pltpu.VMEM((n, D1), jnp.float32),
        ],
    )(tok_tm, inst_tm,
      p['Wih0f'], p['Whh0f'], b0f,
      p['Wih0b'], p['Whh0b'], b0b,
      p['Wih1f'], p['Whh1f'], b1f,
      p['Wih1b'], p['Whh1b'], b1b,
      p['Wfc'], p['bfc'].reshape(1, NUM_CLASSES))


def kernel(x, edge_index, local_ids, global_ids, token_adj, token_embs, params):
    p = params
    src = edge_index[0]
    dst = edge_index[1]
    al1f = p['al1'].reshape(HEADS * HID, 1)
    ar1f = p['ar1'].reshape(HEADS * HID, 1)

    h1pre, el1, er1, pg = _prep1(x, p['W1'], al1f, ar1f, token_embs, p['Wg1'])

    a1, ssrc, sdst, cnts = _gat_sc_1(
        el1.reshape(-1), er1.reshape(-1), src, dst)
    y = _gcn_b(token_adj, pg, p['Wg2'])

    h2pre, el2, er2 = _prep2(a1.reshape(HEADS, N_NODES, N_NODES), h1pre,
                             p['W2'],
                             p['al2'].reshape(OUT_DIM, 1),
                             p['ar2'].reshape(OUT_DIM, 1))

    a2 = _gat_sc_2(el2.reshape(-1), er2.reshape(-1), ssrc, sdst, cnts)
    if isinstance(a2, (tuple, list)):
        a2 = a2[0]
    t, h2 = _gcn_c(token_adj, y, a2.reshape(N_NODES, N_NODES), h2pre)

    tok_tm, inst_tm = _gather_sc(t, h2, local_ids.reshape(-1),
                                 global_ids.reshape(-1))

    return _lstm(tok_tm, inst_tm, p)


# speedup vs baseline: 1.0435x; 1.0435x over previous
import functools
import jax
import jax.numpy as jnp
from jax import lax
from jax.experimental import pallas as pl
from jax.experimental.pallas import tpu as pltpu
from jax.experimental.pallas import tpu_sc as plsc

N_NODES = 2048
N_EDGES = 65536
N_TOKENS = 4096
IN_DIM = 128
HID = 64
HEADS = 4
OUT_DIM = 128
BB = 16
NODE_COUNT = 128
NUM_CLASSES = 16
LSTM_H = 100

NC = 2
NS = 16
NW = NC * NS


def _prep1_body(x_ref, w1_ref, al_ref, ar_ref, tok_ref, wg1_ref,
                h_ref, el_ref, er_ref, p_ref):
    h = jnp.dot(x_ref[...], w1_ref[...], preferred_element_type=jnp.float32)
    h_ref[...] = h
    row = lax.broadcasted_iota(jnp.int32, (HEADS * HID, HEADS), 0)
    col = lax.broadcasted_iota(jnp.int32, (HEADS * HID, HEADS), 1)
    sel = (row // HID) == col
    amat = jnp.where(sel, al_ref[...].reshape(HEADS * HID, 1), 0.0)
    bmat = jnp.where(sel, ar_ref[...].reshape(HEADS * HID, 1), 0.0)
    el_ref[...] = jnp.dot(h, amat, preferred_element_type=jnp.float32)
    er_ref[...] = jnp.dot(h, bmat, preferred_element_type=jnp.float32)
    p_ref[...] = jnp.dot(tok_ref[...], wg1_ref[...],
                         preferred_element_type=jnp.float32)


def _prep1(x, w1, al1f, ar1f, tok, wg1):
    return pl.pallas_call(
        _prep1_body,
        out_shape=(
            jax.ShapeDtypeStruct((N_NODES, HEADS * HID), jnp.float32),
            jax.ShapeDtypeStruct((N_NODES, HEADS), jnp.float32),
            jax.ShapeDtypeStruct((N_NODES, HEADS), jnp.float32),
            jax.ShapeDtypeStruct((N_TOKENS, HID), jnp.float32),
        ),
    )(x, w1, al1f, ar1f, tok, wg1)


_CAP = 4096
_SCAN_BLK = 4096
_ROWS = N_NODES // NW


def _make_gat_scatter_sc(n_heads, pass_rows, with_scan):
    n_pass = _ROWS // pass_rows
    pn = pass_rows * N_NODES
    nh_pn = n_heads * pn

    mesh = plsc.VectorSubcoreMesh(core_axis_name="c", subcore_axis_name="s",
                                  num_cores=NC, num_subcores=NS)

    out_type = [
        jax.ShapeDtypeStruct((n_heads * N_NODES * N_NODES,), jnp.float32),
    ]
    scratch = [
        pltpu.VMEM((N_NODES * n_heads,), jnp.float32),
        pltpu.VMEM((N_NODES * n_heads,), jnp.float32),
        pltpu.VMEM((_CAP,), jnp.int32),
        pltpu.VMEM((_CAP,), jnp.int32),
        pltpu.VMEM((16,), jnp.int32),
        pltpu.VMEM((n_heads * _CAP,), jnp.float32),
        pltpu.VMEM((2 * nh_pn,), jnp.float32),
        pltpu.SemaphoreType.DMA((2,)),
    ]
    if with_scan:
        out_type += [
            jax.ShapeDtypeStruct((NW * _CAP,), jnp.int32),
            jax.ShapeDtypeStruct((NW * _CAP,), jnp.int32),
            jax.ShapeDtypeStruct((NW * 16,), jnp.int32),
        ]
        scratch += [
            pltpu.VMEM((2 * _SCAN_BLK,), jnp.int32),
            pltpu.VMEM((2 * _SCAN_BLK,), jnp.int32),
            pltpu.SemaphoreType.DMA((2,)),
            pltpu.SemaphoreType.DMA((2,)),
        ]

    @functools.partial(
        pl.kernel,
        out_type=tuple(out_type),
        mesh=mesh,
        compiler_params=pltpu.CompilerParams(needs_layout_passes=False),
        scratch_types=scratch,
    )
    def gat_kernel(el_hbm, er_hbm, sa_hbm, da_hbm, *rest):
        if with_scan:
            (a_hbm, ssrc_hbm, sdst_hbm, cnt_hbm,
             el_v, er_v, sel_src, sel_dst, cbuf, w_v, a_v, sem_a,
             blk_src, blk_dst, sem_s, sem_d) = rest
        else:
            (cn_hbm, a_hbm,
             el_v, er_v, sel_src, sel_dst, cbuf, w_v, a_v, sem_a) = rest
        cid = lax.axis_index("c")
        sid = lax.axis_index("s")
        wid = cid * NS + sid
        iota16 = lax.iota(jnp.int32, 16)
        zero16 = jnp.zeros((16,), jnp.float32)

        pltpu.sync_copy(el_hbm, el_v)
        pltpu.sync_copy(er_hbm, er_v)

        if with_scan:
            def zsel(i, carry):
                sel_src[pl.ds(i * 16, 16)] = jnp.zeros((16,), jnp.int32)
                sel_dst[pl.ds(i * 16, 16)] = jnp.zeros((16,), jnp.int32)
                return carry
            lax.fori_loop(0, _CAP // 16, zsel, 0)
            n_blk = N_EDGES // _SCAN_BLK

            def issue_blk(b):
                par = (b % 2) * _SCAN_BLK
                pltpu.async_copy(
                    sa_hbm.at[pl.ds(b * _SCAN_BLK, _SCAN_BLK)],
                    blk_src.at[pl.ds(par, _SCAN_BLK)], sem_s.at[b % 2])
                pltpu.async_copy(
                    da_hbm.at[pl.ds(b * _SCAN_BLK, _SCAN_BLK)],
                    blk_dst.at[pl.ds(par, _SCAN_BLK)], sem_d.at[b % 2])

            issue_blk(0)
            cnt = jnp.int32(0)
            for blk in range(n_blk):
                if blk + 1 < n_blk:
                    issue_blk(blk + 1)
                par = (blk % 2) * _SCAN_BLK
                pltpu.make_async_copy(
                    sa_hbm.at[pl.ds(blk * _SCAN_BLK, _SCAN_BLK)],
                    blk_src.at[pl.ds(par, _SCAN_BLK)],
                    sem_s.at[blk % 2]).wait()
                pltpu.make_async_copy(
                    da_hbm.at[pl.ds(blk * _SCAN_BLK, _SCAN_BLK)],
                    blk_dst.at[pl.ds(par, _SCAN_BLK)],
                    sem_d.at[blk % 2]).wait()

                def scan_body(ci, off, par=par):
                    sv = blk_src[pl.ds(par + ci * 16, 16)]
                    dv = blk_dst[pl.ds(par + ci * 16, 16)]
                    m = (dv >> 6) == wid
                    plsc.store_compressed(sel_src.at[pl.ds(off, 16)], sv,
                                          mask=m)
                    plsc.store_compressed(sel_dst.at[pl.ds(off, 16)], dv,
                                          mask=m)
                    nsel = plsc.all_reduce_population_count(m)
                    return off + nsel[0]

                cnt = lax.fori_loop(0, _SCAN_BLK // 16, scan_body, cnt)
            cbuf[...] = jnp.full((16,), cnt, jnp.int32)
            pltpu.sync_copy(cbuf, cnt_hbm.at[pl.ds(wid * 16, 16)])
            pltpu.sync_copy(sel_src, ssrc_hbm.at[pl.ds(wid * _CAP, _CAP)])
            pltpu.sync_copy(sel_dst, sdst_hbm.at[pl.ds(wid * _CAP, _CAP)])
        else:
            pltpu.sync_copy(sa_hbm.at[pl.ds(wid * _CAP, _CAP)], sel_src)
            pltpu.sync_copy(da_hbm.at[pl.ds(wid * _CAP, _CAP)], sel_dst)
            pltpu.sync_copy(cn_hbm.at[pl.ds(wid * 16, 16)], cbuf)
            cnt = cbuf[pl.ds(0, 16)][0]

        nc_chunks = (cnt + 15) >> 4

        def wpass(ci, carry):
            sv = sel_src[pl.ds(ci * 16, 16)] & (N_NODES - 1)
            dv = sel_dst[pl.ds(ci * 16, 16)] & (N_NODES - 1)
            mv = (ci * 16 + iota16) < cnt
            for h in range(n_heads):
                elh = plsc.load_gather(el_v, [sv * n_heads + h])
                erh = plsc.load_gather(er_v, [dv * n_heads + h])
                e = elh + erh
                e = jnp.where(e >= 0.0, e, 0.2 * e)
                w = jnp.where(mv, jnp.exp(e), 0.0)
                w_v[pl.ds(h * _CAP + ci * 16, 16)] = w
            return carry

        lax.fori_loop(0, nc_chunks, wpass, 0)

        def wb_copy(q, h):
            par = (q % 2) * nh_pn
            off = (h * N_NODES + wid * _ROWS + q * pass_rows) * N_NODES
            return pltpu.make_async_copy(a_v.at[pl.ds(par + h * pn, pn)],
                                         a_hbm.at[pl.ds(off, pn)],
                                         sem_a.at[q % 2])

        for q in range(n_pass):
            par = (q % 2) * nh_pn
            if q >= 2:
                for h in range(n_heads):
                    wb_copy(q - 2, h).wait()

            def zloop(i, carry, par=par):
                for u in range(8):
                    a_v[pl.ds(par + (i * 8 + u) * 16, 16)] = zero16
                return carry
            lax.fori_loop(0, nh_pn // 128, zloop, 0)

            def spass(ci, carry, q=q, par=par):
                sv = sel_src[pl.ds(ci * 16, 16)] & (N_NODES - 1)
                dv = sel_dst[pl.ds(ci * 16, 16)]
                if n_pass > 1:
                    pr_shift = pass_rows.bit_length() - 1
                    mq = ((dv >> pr_shift) & (n_pass - 1)) == q
                else:
                    mq = None
                idx = (dv & (pass_rows - 1)) * N_NODES + sv + par
                for h in range(n_heads):
                    w = w_v[pl.ds(h * _CAP + ci * 16, 16)]
                    plsc.addupdate_scatter(a_v, [idx + h * pn], w, mask=mq)
                return carry

            lax.fori_loop(0, nc_chunks, spass, 0)

            for h in range(n_heads):
                par2 = (q % 2) * nh_pn
                off = (h * N_NODES + wid * _ROWS + q * pass_rows) * N_NODES
                pltpu.async_copy(a_v.at[pl.ds(par2 + h * pn, pn)],
                                 a_hbm.at[pl.ds(off, pn)],
                                 sem_a.at[q % 2])

        for q in range(max(0, n_pass - 2), n_pass):
            for h in range(n_heads):
                wb_copy(q, h).wait()

    return gat_kernel


_gat_sc_1 = _make_gat_scatter_sc(HEADS, 4, True)
_gat_sc_2 = _make_gat_scatter_sc(1, 16, False)


_BLK = 256


def _prep2_body(a_ref, h_ref, w2_ref, al_ref, ar_ref,
                h2_ref, el_ref, er_ref):
    parts = []
    for h in range(HEADS):
        a = a_ref[h]
        num = jnp.dot(a, h_ref[:, h * HID:(h + 1) * HID],
                      preferred_element_type=jnp.float32)
        den = jnp.sum(a, axis=1, keepdims=True) + 1e-9
        parts.append(num / den)
    h1 = jnp.maximum(jnp.concatenate(parts, axis=1), 0.0)
    h2 = jnp.dot(h1, w2_ref[...], preferred_element_type=jnp.float32)
    h2_ref[...] = h2
    el_ref[...] = jnp.dot(h2, al_ref[...], preferred_element_type=jnp.float32)
    er_ref[...] = jnp.dot(h2, ar_ref[...], preferred_element_type=jnp.float32)


def _prep2(a1, h1pre, w2, al2f, ar2f):
    nb = N_NODES // _BLK
    return pl.pallas_call(
        _prep2_body,
        grid=(nb,),
        in_specs=[
            pl.BlockSpec((HEADS, _BLK, N_NODES), lambda i: (0, i, 0)),
            pl.BlockSpec((N_NODES, HEADS * HID), lambda i: (0, 0)),
            pl.BlockSpec((HEADS * HID, OUT_DIM), lambda i: (0, 0)),
            pl.BlockSpec((OUT_DIM, 1), lambda i: (0, 0)),
            pl.BlockSpec((OUT_DIM, 1), lambda i: (0, 0)),
        ],
        out_specs=(
            pl.BlockSpec((_BLK, OUT_DIM), lambda i: (i, 0)),
            pl.BlockSpec((_BLK, 1), lambda i: (i, 0)),
            pl.BlockSpec((_BLK, 1), lambda i: (i, 0)),
        ),
        out_shape=(
            jax.ShapeDtypeStruct((N_NODES, OUT_DIM), jnp.float32),
            jax.ShapeDtypeStruct((N_NODES, 1), jnp.float32),
            jax.ShapeDtypeStruct((N_NODES, 1), jnp.float32),
        ),
    )(a1, h1pre, w2, al2f, ar2f)


_GCN_BLK = 512


def _gcn_b_body(adj_ref, p_ref, wg2_ref, y_ref):
    acc = jnp.dot(adj_ref[...], p_ref[...], preferred_element_type=jnp.float32)
    y_ref[...] = jnp.dot(jnp.maximum(acc, 0.0), wg2_ref[...],
                         preferred_element_type=jnp.float32)


def _gcn_b(adj, p, wg2):
    nb = N_TOKENS // _GCN_BLK
    return pl.pallas_call(
        _gcn_b_body,
        grid=(nb,),
        in_specs=[
            pl.BlockSpec((_GCN_BLK, N_TOKENS), lambda i: (i, 0)),
            pl.BlockSpec((N_TOKENS, HID), lambda i: (0, 0)),
            pl.BlockSpec((HID, OUT_DIM), lambda i: (0, 0)),
        ],
        out_specs=pl.BlockSpec((_GCN_BLK, OUT_DIM), lambda i: (i, 0)),
        out_shape=jax.ShapeDtypeStruct((N_TOKENS, OUT_DIM), jnp.float32),
    )(adj, p, wg2)


def _gcn_c_body(adj_ref, y_ref, a2_ref, h2p_ref, t_ref, h2_ref):
    t_ref[...] = jnp.dot(adj_ref[...], y_ref[...],
                         preferred_element_type=jnp.float32)
    a = a2_ref[...]
    num = jnp.dot(a, h2p_ref[...], preferred_element_type=jnp.float32)
    den = jnp.sum(a, axis=1, keepdims=True) + 1e-9
    h2_ref[...] = num / den


def _gcn_c(adj, y, a2, h2pre):
    nb = N_TOKENS // _GCN_BLK
    hblk = N_NODES // nb
    return pl.pallas_call(
        _gcn_c_body,
        grid=(nb,),
        in_specs=[
            pl.BlockSpec((_GCN_BLK, N_TOKENS), lambda i: (i, 0)),
            pl.BlockSpec((N_TOKENS, OUT_DIM), lambda i: (0, 0)),
            pl.BlockSpec((hblk, N_NODES), lambda i: (i, 0)),
            pl.BlockSpec((N_NODES, OUT_DIM), lambda i: (0, 0)),
        ],
        out_specs=(
            pl.BlockSpec((_GCN_BLK, OUT_DIM), lambda i: (i, 0)),
            pl.BlockSpec((hblk, OUT_DIM), lambda i: (i, 0)),
        ),
        out_shape=(
            jax.ShapeDtypeStruct((N_TOKENS, OUT_DIM), jnp.float32),
            jax.ShapeDtypeStruct((N_NODES, OUT_DIM), jnp.float32),
        ),
    )(adj, y, a2, h2pre)


def _make_gather_sc():
    mesh = plsc.VectorSubcoreMesh(core_axis_name="c", subcore_axis_name="s",
                                  num_cores=NC, num_subcores=NS)
    t_per_tile = NODE_COUNT // NW

    @functools.partial(
        pl.kernel,
        out_type=(
            jax.ShapeDtypeStruct((BB * NODE_COUNT, OUT_DIM), jnp.float32),
            jax.ShapeDtypeStruct((BB * NODE_COUNT, OUT_DIM), jnp.float32),
        ),
        mesh=mesh,
        compiler_params=pltpu.CompilerParams(needs_layout_passes=False),
        scratch_types=[
            pltpu.VMEM((BB * NODE_COUNT,), jnp.int32),
            pltpu.VMEM((BB * NODE_COUNT,), jnp.int32),
            pltpu.VMEM((16, OUT_DIM), jnp.float32),
            pltpu.VMEM((16, OUT_DIM), jnp.float32),
            pltpu.SemaphoreType.DMA,
            pltpu.SemaphoreType.DMA,
        ],
    )
    def gather_kernel(t_hbm, h2_hbm, lids_hbm, gids_hbm,
                      tok_out, inst_out,
                      lids_v, gids_v, rows_a, rows_b, sem_a, sem_b):
        cid = lax.axis_index("c")
        sid = lax.axis_index("s")
        wid = sid * NC + cid
        iota16 = lax.iota(jnp.int32, 16)
        pltpu.sync_copy(lids_hbm, lids_v)
        pltpu.sync_copy(gids_hbm, gids_v)

        def body(i, carry):
            t = wid * t_per_tile + i
            flat = iota16 * NODE_COUNT + t
            gv = plsc.load_gather(gids_v, [flat])
            ca = pltpu.async_copy(t_hbm.at[gv], rows_a, sem_a)
            lv = plsc.load_gather(lids_v, [flat])
            inst_idx = iota16 * NODE_COUNT + lv
            cb = pltpu.async_copy(h2_hbm.at[inst_idx], rows_b, sem_b)
            ca.wait()
            pltpu.sync_copy(rows_a, tok_out.at[pl.ds(t * 16, 16)])
            cb.wait()
            pltpu.sync_copy(rows_b, inst_out.at[pl.ds(t * 16, 16)])
            return carry

        lax.fori_loop(0, t_per_tile, body, 0)

    return gather_kernel


_gather_sc = _make_gather_sc()


def _lstm_body(tok_ref, inst_ref,
               wih0f_ref, whh0f_ref, b0f_ref,
               wih0b_ref, whh0b_ref, b0b_ref,
               wih1f_ref, whh1f_ref, b1f_ref,
               wih1b_ref, whh1b_ref, b1b_ref,
               wfc_ref, bfc_ref,
               out_ref,
               xga_ref, xgb_ref, xgcat_ref, l0_ref):
    cdims = (((1,), (1,)), ((), ()))
    T = NODE_COUNT
    H = LSTM_H
    z100 = jnp.zeros((H, H), jnp.float32)

    def whh_cat(wf, wb):
        blocks = []
        for g in range(4):
            blocks.append(jnp.concatenate([wf[g * H:(g + 1) * H], z100], 1))
            blocks.append(jnp.concatenate([z100, wb[g * H:(g + 1) * H]], 1))
        return jnp.concatenate(blocks, 0)

    def interleave(k, carry):
        kb = T - 1 - k
        xf = xga_ref[pl.ds(k * BB, BB), :]
        xb = xgb_ref[pl.ds(kb * BB, BB), :]
        parts = []
        for g in range(4):
            parts.append(xf[:, g * H:(g + 1) * H])
            parts.append(xb[:, g * H:(g + 1) * H])
        xgcat_ref[pl.ds(k * BB, BB), :] = jnp.concatenate(parts, 1)
        return carry

    def packed_cell(g, ccat):
        i = jax.nn.sigmoid(g[:, :2 * H])
        f = jax.nn.sigmoid(g[:, 2 * H:4 * H])
        gg = jnp.tanh(g[:, 4 * H:6 * H])
        o = jax.nn.sigmoid(g[:, 6 * H:])
        c2 = f * ccat + i * gg
        h2 = o * jnp.tanh(c2)
        return h2, c2

    tok = tok_ref[...]
    inst = inst_ref[...]

    def xgates0(w_ref, b_ref):
        w = w_ref[...]
        g = lax.dot_general(tok, w[:, :OUT_DIM], cdims,
                            preferred_element_type=jnp.float32)
        g = g + lax.dot_general(inst, w[:, OUT_DIM:], cdims,
                                preferred_element_type=jnp.float32)
        return g + b_ref[...]

    xga_ref[...] = xgates0(wih0f_ref, b0f_ref)
    xgb_ref[...] = xgates0(wih0b_ref, b0b_ref)
    lax.fori_loop(0, T, interleave, 0, unroll=8)
    w0 = whh_cat(whh0f_ref[...], whh0b_ref[...])

    def step0(k, carry):
        hcat, ccat = carry
        g = xgcat_ref[pl.ds(k * BB, BB), :] + lax.dot_general(
            hcat, w0, (((1,), (1,)), ((), ())),
            preferred_element_type=jnp.float32)
        hcat, ccat = packed_cell(g, ccat)
        kb = T - 1 - k
        l0_ref[pl.ds(k * BB, BB), :H] = hcat[:, :H]
        l0_ref[pl.ds(kb * BB, BB), H:] = hcat[:, H:]
        return hcat, ccat

    z = jnp.zeros((BB, 2 * H), jnp.float32)
    lax.fori_loop(0, T, step0, (z, z), unroll=8)

    l0 = l0_ref[...]
    xga_ref[...] = lax.dot_general(l0, wih1f_ref[...], cdims,
                                   preferred_element_type=jnp.float32) + b1f_ref[...]
    xgb_ref[...] = lax.dot_general(l0, wih1b_ref[...], cdims,
                                   preferred_element_type=jnp.float32) + b1b_ref[...]
    lax.fori_loop(0, T, interleave, 0, unroll=8)
    w1 = whh_cat(whh1f_ref[...], whh1b_ref[...])

    def step1(k, carry):
        hcat, ccat = carry
        g = xgcat_ref[pl.ds(k * BB, BB), :] + lax.dot_general(
            hcat, w1, (((1,), (1,)), ((), ())),
            preferred_element_type=jnp.float32)
        return packed_cell(g, ccat)

    hcat, _ = lax.fori_loop(0, T, step1, (z, z), unroll=8)
    out_ref[...] = jnp.dot(hcat, wfc_ref[...],
                           preferred_element_type=jnp.float32) + bfc_ref[...]


def _lstm(tok_tm, inst_tm, p):
    D1 = 2 * LSTM_H
    n = BB * NODE_COUNT
    b0f = (p['bih0f'] + p['bhh0f']).reshape(1, 4 * LSTM_H)
    b0b = (p['bih0b'] + p['bhh0b']).reshape(1, 4 * LSTM_H)
    b1f = (p['bih1f'] + p['bhh1f']).reshape(1, 4 * LSTM_H)
    b1b = (p['bih1b'] + p['bhh1b']).reshape(1, 4 * LSTM_H)
    return pl.pallas_call(
        _lstm_body,
        out_shape=jax.ShapeDtypeStruct((BB, NUM_CLASSES), jnp.float32),
        scratch_shapes=[
            pltpu.VMEM((n, 4 * LSTM_H), jnp.float32),
            pltpu.VMEM((n, 4 * LSTM_H), jnp.float32),
            pltpu.VMEM((n, 8 * LSTM_H), jnp.float32),
            pltpu.VMEM((n, D1), jnp.float32),
        ],
    )(tok_tm, inst_tm,
      p['Wih0f'], p['Whh0f'], b0f,
      p['Wih0b'], p['Whh0b'], b0b,
      p['Wih1f'], p['Whh1f'], b1f,
      p['Wih1b'], p['Whh1b'], b1b,
      p['Wfc'], p['bfc'].reshape(1, NUM_CLASSES))


def kernel(x, edge_index, local_ids, global_ids, token_adj, token_embs, params):
    p = params
    src = edge_index[0]
    dst = edge_index[1]
    al1f = p['al1'].reshape(HEADS * HID, 1)
    ar1f = p['ar1'].reshape(HEADS * HID, 1)

    h1pre, el1, er1, pg = _prep1(x, p['W1'], al1f, ar1f, token_embs, p['Wg1'])

    a1, ssrc, sdst, cnts = _gat_sc_1(
        el1.reshape(-1), er1.reshape(-1), src, dst)
    y = _gcn_b(token_adj, pg, p['Wg2'])

    h2pre, el2, er2 = _prep2(a1.reshape(HEADS, N_NODES, N_NODES), h1pre,
                             p['W2'],
                             p['al2'].reshape(OUT_DIM, 1),
                             p['ar2'].reshape(OUT_DIM, 1))

    a2 = _gat_sc_2(el2.reshape(-1), er2.reshape(-1), ssrc, sdst, cnts)
    if isinstance(a2, (tuple, list)):
        a2 = a2[0]
    t, h2 = _gcn_c(token_adj, y, a2.reshape(N_NODES, N_NODES), h2pre)

    tok_tm, inst_tm = _gather_sc(t, h2, local_ids.reshape(-1),
                                 global_ids.reshape(-1))

    return _lstm(tok_tm, inst_tm, p)


# speedup vs baseline: 1.0557x; 1.0116x over previous
import functools
import jax
import jax.numpy as jnp
from jax import lax
from jax.experimental import pallas as pl
from jax.experimental.pallas import tpu as pltpu
from jax.experimental.pallas import tpu_sc as plsc

N_NODES = 2048
N_EDGES = 65536
N_TOKENS = 4096
IN_DIM = 128
HID = 64
HEADS = 4
OUT_DIM = 128
BB = 16
NODE_COUNT = 128
NUM_CLASSES = 16
LSTM_H = 100

NC = 2
NS = 16
NW = NC * NS


def _prep1_body(x_ref, w1_ref, al_ref, ar_ref, tok_ref, wg1_ref,
                h_ref, el_ref, er_ref, p_ref):
    h = jnp.dot(x_ref[...], w1_ref[...], preferred_element_type=jnp.float32)
    h_ref[...] = h
    row = lax.broadcasted_iota(jnp.int32, (HEADS * HID, HEADS), 0)
    col = lax.broadcasted_iota(jnp.int32, (HEADS * HID, HEADS), 1)
    sel = (row // HID) == col
    amat = jnp.where(sel, al_ref[...].reshape(HEADS * HID, 1), 0.0)
    bmat = jnp.where(sel, ar_ref[...].reshape(HEADS * HID, 1), 0.0)
    el_ref[...] = jnp.dot(h, amat, preferred_element_type=jnp.float32)
    er_ref[...] = jnp.dot(h, bmat, preferred_element_type=jnp.float32)
    p_ref[...] = jnp.dot(tok_ref[...], wg1_ref[...],
                         preferred_element_type=jnp.float32)


def _prep1(x, w1, al1f, ar1f, tok, wg1):
    return pl.pallas_call(
        _prep1_body,
        out_shape=(
            jax.ShapeDtypeStruct((N_NODES, HEADS * HID), jnp.float32),
            jax.ShapeDtypeStruct((N_NODES, HEADS), jnp.float32),
            jax.ShapeDtypeStruct((N_NODES, HEADS), jnp.float32),
            jax.ShapeDtypeStruct((N_TOKENS, HID), jnp.float32),
        ),
    )(x, w1, al1f, ar1f, tok, wg1)


_CAP = 4096
_SCAN_BLK = 4096
_ROWS = N_NODES // NW


def _make_gat_scatter_sc(n_heads, pass_rows, with_scan):
    n_pass = _ROWS // pass_rows
    pn = pass_rows * N_NODES
    nh_pn = n_heads * pn

    mesh = plsc.VectorSubcoreMesh(core_axis_name="c", subcore_axis_name="s",
                                  num_cores=NC, num_subcores=NS)

    out_type = [
        jax.ShapeDtypeStruct((n_heads * N_NODES * N_NODES,), jnp.float32),
    ]
    scratch = [
        pltpu.VMEM((N_NODES * n_heads,), jnp.float32),
        pltpu.VMEM((N_NODES * n_heads,), jnp.float32),
        pltpu.VMEM((_CAP,), jnp.int32),
        pltpu.VMEM((_CAP,), jnp.int32),
        pltpu.VMEM((16,), jnp.int32),
        pltpu.VMEM((n_heads * _CAP,), jnp.float32),
        pltpu.VMEM((2 * nh_pn,), jnp.float32),
        pltpu.SemaphoreType.DMA((2,)),
    ]
    if with_scan:
        out_type += [
            jax.ShapeDtypeStruct((NW * _CAP,), jnp.int32),
            jax.ShapeDtypeStruct((NW * _CAP,), jnp.int32),
            jax.ShapeDtypeStruct((NW * 16,), jnp.int32),
        ]
        scratch += [
            pltpu.VMEM((2 * _SCAN_BLK,), jnp.int32),
            pltpu.VMEM((2 * _SCAN_BLK,), jnp.int32),
            pltpu.SemaphoreType.DMA((2,)),
            pltpu.SemaphoreType.DMA((2,)),
        ]

    @functools.partial(
        pl.kernel,
        out_type=tuple(out_type),
        mesh=mesh,
        compiler_params=pltpu.CompilerParams(needs_layout_passes=False),
        scratch_types=scratch,
    )
    def gat_kernel(el_hbm, er_hbm, sa_hbm, da_hbm, *rest):
        if with_scan:
            (a_hbm, ssrc_hbm, sdst_hbm, cnt_hbm,
             el_v, er_v, sel_src, sel_dst, cbuf, w_v, a_v, sem_a,
             blk_src, blk_dst, sem_s, sem_d) = rest
        else:
            (cn_hbm, a_hbm,
             el_v, er_v, sel_src, sel_dst, cbuf, w_v, a_v, sem_a) = rest
        cid = lax.axis_index("c")
        sid = lax.axis_index("s")
        wid = cid * NS + sid
        iota16 = lax.iota(jnp.int32, 16)
        zero16 = jnp.zeros((16,), jnp.float32)

        pltpu.sync_copy(el_hbm, el_v)
        pltpu.sync_copy(er_hbm, er_v)

        if with_scan:
            def zsel(i, carry):
                sel_src[pl.ds(i * 16, 16)] = jnp.zeros((16,), jnp.int32)
                sel_dst[pl.ds(i * 16, 16)] = jnp.zeros((16,), jnp.int32)
                return carry
            lax.fori_loop(0, _CAP // 16, zsel, 0, unroll=4)
            n_blk = N_EDGES // _SCAN_BLK

            def issue_blk(b):
                par = (b % 2) * _SCAN_BLK
                pltpu.async_copy(
                    sa_hbm.at[pl.ds(b * _SCAN_BLK, _SCAN_BLK)],
                    blk_src.at[pl.ds(par, _SCAN_BLK)], sem_s.at[b % 2])
                pltpu.async_copy(
                    da_hbm.at[pl.ds(b * _SCAN_BLK, _SCAN_BLK)],
                    blk_dst.at[pl.ds(par, _SCAN_BLK)], sem_d.at[b % 2])

            issue_blk(0)
            cnt = jnp.int32(0)
            for blk in range(n_blk):
                if blk + 1 < n_blk:
                    issue_blk(blk + 1)
                par = (blk % 2) * _SCAN_BLK
                pltpu.make_async_copy(
                    sa_hbm.at[pl.ds(blk * _SCAN_BLK, _SCAN_BLK)],
                    blk_src.at[pl.ds(par, _SCAN_BLK)],
                    sem_s.at[blk % 2]).wait()
                pltpu.make_async_copy(
                    da_hbm.at[pl.ds(blk * _SCAN_BLK, _SCAN_BLK)],
                    blk_dst.at[pl.ds(par, _SCAN_BLK)],
                    sem_d.at[blk % 2]).wait()

                def scan_body(ci, off, par=par):
                    sv = blk_src[pl.ds(par + ci * 16, 16)]
                    dv = blk_dst[pl.ds(par + ci * 16, 16)]
                    m = (dv >> 6) == wid
                    plsc.store_compressed(sel_src.at[pl.ds(off, 16)], sv,
                                          mask=m)
                    plsc.store_compressed(sel_dst.at[pl.ds(off, 16)], dv,
                                          mask=m)
                    nsel = plsc.all_reduce_population_count(m)
                    return off + nsel[0]

                cnt = lax.fori_loop(0, _SCAN_BLK // 16, scan_body, cnt,
                                    unroll=4)
            cbuf[...] = jnp.full((16,), cnt, jnp.int32)
            pltpu.sync_copy(cbuf, cnt_hbm.at[pl.ds(wid * 16, 16)])
            pltpu.sync_copy(sel_src, ssrc_hbm.at[pl.ds(wid * _CAP, _CAP)])
            pltpu.sync_copy(sel_dst, sdst_hbm.at[pl.ds(wid * _CAP, _CAP)])
        else:
            pltpu.sync_copy(sa_hbm.at[pl.ds(wid * _CAP, _CAP)], sel_src)
            pltpu.sync_copy(da_hbm.at[pl.ds(wid * _CAP, _CAP)], sel_dst)
            pltpu.sync_copy(cn_hbm.at[pl.ds(wid * 16, 16)], cbuf)
            cnt = cbuf[pl.ds(0, 16)][0]

        nc_chunks = (cnt + 15) >> 4

        def wpass(ci, carry):
            sv = sel_src[pl.ds(ci * 16, 16)] & (N_NODES - 1)
            dv = sel_dst[pl.ds(ci * 16, 16)] & (N_NODES - 1)
            mv = (ci * 16 + iota16) < cnt
            for h in range(n_heads):
                elh = plsc.load_gather(el_v, [sv * n_heads + h])
                erh = plsc.load_gather(er_v, [dv * n_heads + h])
                e = elh + erh
                e = jnp.where(e >= 0.0, e, 0.2 * e)
                w = jnp.where(mv, jnp.exp(e), 0.0)
                w_v[pl.ds(h * _CAP + ci * 16, 16)] = w
            return carry

        lax.fori_loop(0, nc_chunks, wpass, 0)

        def wb_copy(q, h):
            par = (q % 2) * nh_pn
            off = (h * N_NODES + wid * _ROWS + q * pass_rows) * N_NODES
            return pltpu.make_async_copy(a_v.at[pl.ds(par + h * pn, pn)],
                                         a_hbm.at[pl.ds(off, pn)],
                                         sem_a.at[q % 2])

        for q in range(n_pass):
            par = (q % 2) * nh_pn
            if q >= 2:
                for h in range(n_heads):
                    wb_copy(q - 2, h).wait()

            def zloop(i, carry, par=par):
                for u in range(8):
                    a_v[pl.ds(par + (i * 8 + u) * 16, 16)] = zero16
                return carry
            lax.fori_loop(0, nh_pn // 128, zloop, 0, unroll=2)

            def spass(ci, carry, q=q, par=par):
                sv = sel_src[pl.ds(ci * 16, 16)] & (N_NODES - 1)
                dv = sel_dst[pl.ds(ci * 16, 16)]
                if n_pass > 1:
                    pr_shift = pass_rows.bit_length() - 1
                    mq = ((dv >> pr_shift) & (n_pass - 1)) == q
                else:
                    mq = None
                idx = (dv & (pass_rows - 1)) * N_NODES + sv + par
                for h in range(n_heads):
                    w = w_v[pl.ds(h * _CAP + ci * 16, 16)]
                    plsc.addupdate_scatter(a_v, [idx + h * pn], w, mask=mq)
                return carry

            lax.fori_loop(0, nc_chunks, spass, 0)

            for h in range(n_heads):
                par2 = (q % 2) * nh_pn
                off = (h * N_NODES + wid * _ROWS + q * pass_rows) * N_NODES
                pltpu.async_copy(a_v.at[pl.ds(par2 + h * pn, pn)],
                                 a_hbm.at[pl.ds(off, pn)],
                                 sem_a.at[q % 2])

        for q in range(max(0, n_pass - 2), n_pass):
            for h in range(n_heads):
                wb_copy(q, h).wait()

    return gat_kernel


_gat_sc_1 = _make_gat_scatter_sc(HEADS, 4, True)
_gat_sc_2 = _make_gat_scatter_sc(1, 16, False)


_BLK = 256


def _prep2_body(a_ref, h_ref, w2_ref, al_ref, ar_ref,
                h2_ref, el_ref, er_ref):
    parts = []
    for h in range(HEADS):
        a = a_ref[h]
        num = jnp.dot(a, h_ref[:, h * HID:(h + 1) * HID],
                      preferred_element_type=jnp.float32)
        den = jnp.sum(a, axis=1, keepdims=True) + 1e-9
        parts.append(num / den)
    h1 = jnp.maximum(jnp.concatenate(parts, axis=1), 0.0)
    h2 = jnp.dot(h1, w2_ref[...], preferred_element_type=jnp.float32)
    h2_ref[...] = h2
    el_ref[...] = jnp.dot(h2, al_ref[...], preferred_element_type=jnp.float32)
    er_ref[...] = jnp.dot(h2, ar_ref[...], preferred_element_type=jnp.float32)


def _prep2(a1, h1pre, w2, al2f, ar2f):
    nb = N_NODES // _BLK
    return pl.pallas_call(
        _prep2_body,
        grid=(nb,),
        in_specs=[
            pl.BlockSpec((HEADS, _BLK, N_NODES), lambda i: (0, i, 0)),
            pl.BlockSpec((N_NODES, HEADS * HID), lambda i: (0, 0)),
            pl.BlockSpec((HEADS * HID, OUT_DIM), lambda i: (0, 0)),
            pl.BlockSpec((OUT_DIM, 1), lambda i: (0, 0)),
            pl.BlockSpec((OUT_DIM, 1), lambda i: (0, 0)),
        ],
        out_specs=(
            pl.BlockSpec((_BLK, OUT_DIM), lambda i: (i, 0)),
            pl.BlockSpec((_BLK, 1), lambda i: (i, 0)),
            pl.BlockSpec((_BLK, 1), lambda i: (i, 0)),
        ),
        out_shape=(
            jax.ShapeDtypeStruct((N_NODES, OUT_DIM), jnp.float32),
            jax.ShapeDtypeStruct((N_NODES, 1), jnp.float32),
            jax.ShapeDtypeStruct((N_NODES, 1), jnp.float32),
        ),
    )(a1, h1pre, w2, al2f, ar2f)


_GCN_BLK = 512


def _gcn_b_body(adj_ref, p_ref, wg2_ref, y_ref):
    acc = jnp.dot(adj_ref[...], p_ref[...], preferred_element_type=jnp.float32)
    y_ref[...] = jnp.dot(jnp.maximum(acc, 0.0), wg2_ref[...],
                         preferred_element_type=jnp.float32)


def _gcn_b(adj, p, wg2):
    nb = N_TOKENS // _GCN_BLK
    return pl.pallas_call(
        _gcn_b_body,
        grid=(nb,),
        in_specs=[
            pl.BlockSpec((_GCN_BLK, N_TOKENS), lambda i: (i, 0)),
            pl.BlockSpec((N_TOKENS, HID), lambda i: (0, 0)),
            pl.BlockSpec((HID, OUT_DIM), lambda i: (0, 0)),
        ],
        out_specs=pl.BlockSpec((_GCN_BLK, OUT_DIM), lambda i: (i, 0)),
        out_shape=jax.ShapeDtypeStruct((N_TOKENS, OUT_DIM), jnp.float32),
    )(adj, p, wg2)


def _gcn_c_body(adj_ref, y_ref, a2_ref, h2p_ref, t_ref, h2_ref):
    t_ref[...] = jnp.dot(adj_ref[...], y_ref[...],
                         preferred_element_type=jnp.float32)
    a = a2_ref[...]
    num = jnp.dot(a, h2p_ref[...], preferred_element_type=jnp.float32)
    den = jnp.sum(a, axis=1, keepdims=True) + 1e-9
    h2_ref[...] = num / den


def _gcn_c(adj, y, a2, h2pre):
    nb = N_TOKENS // _GCN_BLK
    hblk = N_NODES // nb
    return pl.pallas_call(
        _gcn_c_body,
        grid=(nb,),
        in_specs=[
            pl.BlockSpec((_GCN_BLK, N_TOKENS), lambda i: (i, 0)),
            pl.BlockSpec((N_TOKENS, OUT_DIM), lambda i: (0, 0)),
            pl.BlockSpec((hblk, N_NODES), lambda i: (i, 0)),
            pl.BlockSpec((N_NODES, OUT_DIM), lambda i: (0, 0)),
        ],
        out_specs=(
            pl.BlockSpec((_GCN_BLK, OUT_DIM), lambda i: (i, 0)),
            pl.BlockSpec((hblk, OUT_DIM), lambda i: (i, 0)),
        ),
        out_shape=(
            jax.ShapeDtypeStruct((N_TOKENS, OUT_DIM), jnp.float32),
            jax.ShapeDtypeStruct((N_NODES, OUT_DIM), jnp.float32),
        ),
    )(adj, y, a2, h2pre)


def _make_gather_sc():
    mesh = plsc.VectorSubcoreMesh(core_axis_name="c", subcore_axis_name="s",
                                  num_cores=NC, num_subcores=NS)
    t_per_tile = NODE_COUNT // NW

    @functools.partial(
        pl.kernel,
        out_type=(
            jax.ShapeDtypeStruct((BB * NODE_COUNT, OUT_DIM), jnp.float32),
            jax.ShapeDtypeStruct((BB * NODE_COUNT, OUT_DIM), jnp.float32),
        ),
        mesh=mesh,
        compiler_params=pltpu.CompilerParams(needs_layout_passes=False),
        scratch_types=[
            pltpu.VMEM((BB * NODE_COUNT,), jnp.int32),
            pltpu.VMEM((BB * NODE_COUNT,), jnp.int32),
            pltpu.VMEM((16, OUT_DIM), jnp.float32),
            pltpu.VMEM((16, OUT_DIM), jnp.float32),
            pltpu.SemaphoreType.DMA,
            pltpu.SemaphoreType.DMA,
        ],
    )
    def gather_kernel(t_hbm, h2_hbm, lids_hbm, gids_hbm,
                      tok_out, inst_out,
                      lids_v, gids_v, rows_a, rows_b, sem_a, sem_b):
        cid = lax.axis_index("c")
        sid = lax.axis_index("s")
        wid = sid * NC + cid
        iota16 = lax.iota(jnp.int32, 16)
        pltpu.sync_copy(lids_hbm, lids_v)
        pltpu.sync_copy(gids_hbm, gids_v)

        def body(i, carry):
            t = wid * t_per_tile + i
            flat = iota16 * NODE_COUNT + t
            gv = plsc.load_gather(gids_v, [flat])
            ca = pltpu.async_copy(t_hbm.at[gv], rows_a, sem_a)
            lv = plsc.load_gather(lids_v, [flat])
            inst_idx = iota16 * NODE_COUNT + lv
            cb = pltpu.async_copy(h2_hbm.at[inst_idx], rows_b, sem_b)
            ca.wait()
            pltpu.sync_copy(rows_a, tok_out.at[pl.ds(t * 16, 16)])
            cb.wait()
            pltpu.sync_copy(rows_b, inst_out.at[pl.ds(t * 16, 16)])
            return carry

        lax.fori_loop(0, t_per_tile, body, 0)

    return gather_kernel


_gather_sc = _make_gather_sc()


def _lstm_body(tok_ref, inst_ref,
               wih0f_ref, whh0f_ref, b0f_ref,
               wih0b_ref, whh0b_ref, b0b_ref,
               wih1f_ref, whh1f_ref, b1f_ref,
               wih1b_ref, whh1b_ref, b1b_ref,
               wfc_ref, bfc_ref,
               out_ref,
               xga_ref, xgb_ref, xgcat_ref, l0_ref):
    cdims = (((1,), (1,)), ((), ()))
    T = NODE_COUNT
    H = LSTM_H
    z100 = jnp.zeros((H, H), jnp.float32)

    def whh_cat(wf, wb):
        blocks = []
        for g in range(4):
            blocks.append(jnp.concatenate([wf[g * H:(g + 1) * H], z100], 1))
            blocks.append(jnp.concatenate([z100, wb[g * H:(g + 1) * H]], 1))
        return jnp.concatenate(blocks, 0)

    def interleave(k, carry):
        kb = T - 1 - k
        xf = xga_ref[pl.ds(k * BB, BB), :]
        xb = xgb_ref[pl.ds(kb * BB, BB), :]
        parts = []
        for g in range(4):
            parts.append(xf[:, g * H:(g + 1) * H])
            parts.append(xb[:, g * H:(g + 1) * H])
        xgcat_ref[pl.ds(k * BB, BB), :] = jnp.concatenate(parts, 1)
        return carry

    def packed_cell(g, ccat):
        i = jax.nn.sigmoid(g[:, :2 * H])
        f = jax.nn.sigmoid(g[:, 2 * H:4 * H])
        gg = jnp.tanh(g[:, 4 * H:6 * H])
        o = jax.nn.sigmoid(g[:, 6 * H:])
        c2 = f * ccat + i * gg
        h2 = o * jnp.tanh(c2)
        return h2, c2

    tok = tok_ref[...]
    inst = inst_ref[...]

    def xgates0(w_ref, b_ref):
        w = w_ref[...]
        g = lax.dot_general(tok, w[:, :OUT_DIM], cdims,
                            preferred_element_type=jnp.float32)
        g = g + lax.dot_general(inst, w[:, OUT_DIM:], cdims,
                                preferred_element_type=jnp.float32)
        return g + b_ref[...]

    xga_ref[...] = xgates0(wih0f_ref, b0f_ref)
    xgb_ref[...] = xgates0(wih0b_ref, b0b_ref)
    lax.fori_loop(0, T, interleave, 0, unroll=8)
    w0 = whh_cat(whh0f_ref[...], whh0b_ref[...])

    def step0(k, carry):
        hcat, ccat = carry
        g = xgcat_ref[pl.ds(k * BB, BB), :] + lax.dot_general(
            hcat, w0, (((1,), (1,)), ((), ())),
            preferred_element_type=jnp.float32)
        hcat, ccat = packed_cell(g, ccat)
        kb = T - 1 - k
        l0_ref[pl.ds(k * BB, BB), :H] = hcat[:, :H]
        l0_ref[pl.ds(kb * BB, BB), H:] = hcat[:, H:]
        return hcat, ccat

    z = jnp.zeros((BB, 2 * H), jnp.float32)
    lax.fori_loop(0, T, step0, (z, z), unroll=8)

    l0 = l0_ref[...]
    xga_ref[...] = lax.dot_general(l0, wih1f_ref[...], cdims,
                                   preferred_element_type=jnp.float32) + b1f_ref[...]
    xgb_ref[...] = lax.dot_general(l0, wih1b_ref[...], cdims,
                                   preferred_element_type=jnp.float32) + b1b_ref[...]
    lax.fori_loop(0, T, interleave, 0, unroll=8)
    w1 = whh_cat(whh1f_ref[...], whh1b_ref[...])

    def step1(k, carry):
        hcat, ccat = carry
        g = xgcat_ref[pl.ds(k * BB, BB), :] + lax.dot_general(
            hcat, w1, (((1,), (1,)), ((), ())),
            preferred_element_type=jnp.float32)
        return packed_cell(g, ccat)

    hcat, _ = lax.fori_loop(0, T, step1, (z, z), unroll=8)
    out_ref[...] = jnp.dot(hcat, wfc_ref[...],
                           preferred_element_type=jnp.float32) + bfc_ref[...]


def _lstm(tok_tm, inst_tm, p):
    D1 = 2 * LSTM_H
    n = BB * NODE_COUNT
    b0f = (p['bih0f'] + p['bhh0f']).reshape(1, 4 * LSTM_H)
    b0b = (p['bih0b'] + p['bhh0b']).reshape(1, 4 * LSTM_H)
    b1f = (p['bih1f'] + p['bhh1f']).reshape(1, 4 * LSTM_H)
    b1b = (p['bih1b'] + p['bhh1b']).reshape(1, 4 * LSTM_H)
    return pl.pallas_call(
        _lstm_body,
        out_shape=jax.ShapeDtypeStruct((BB, NUM_CLASSES), jnp.float32),
        scratch_shapes=[
            pltpu.VMEM((n, 4 * LSTM_H), jnp.float32),
            pltpu.VMEM((n, 4 * LSTM_H), jnp.float32),
            pltpu.VMEM((n, 8 * LSTM_H), jnp.float32),
            pltpu.VMEM((n, D1), jnp.float32),
        ],
    )(tok_tm, inst_tm,
      p['Wih0f'], p['Whh0f'], b0f,
      p['Wih0b'], p['Whh0b'], b0b,
      p['Wih1f'], p['Whh1f'], b1f,
      p['Wih1b'], p['Whh1b'], b1b,
      p['Wfc'], p['bfc'].reshape(1, NUM_CLASSES))


def kernel(x, edge_index, local_ids, global_ids, token_adj, token_embs, params):
    p = params
    src = edge_index[0]
    dst = edge_index[1]
    al1f = p['al1'].reshape(HEADS * HID, 1)
    ar1f = p['ar1'].reshape(HEADS * HID, 1)

    h1pre, el1, er1, pg = _prep1(x, p['W1'], al1f, ar1f, token_embs, p['Wg1'])

    a1, ssrc, sdst, cnts = _gat_sc_1(
        el1.reshape(-1), er1.reshape(-1), src, dst)
    y = _gcn_b(token_adj, pg, p['Wg2'])

    h2pre, el2, er2 = _prep2(a1.reshape(HEADS, N_NODES, N_NODES), h1pre,
                             p['W2'],
                             p['al2'].reshape(OUT_DIM, 1),
                             p['ar2'].reshape(OUT_DIM, 1))

    a2 = _gat_sc_2(el2.reshape(-1), er2.reshape(-1), ssrc, sdst, cnts)
    if isinstance(a2, (tuple, list)):
        a2 = a2[0]
    t, h2 = _gcn_c(token_adj, y, a2.reshape(N_NODES, N_NODES), h2pre)

    tok_tm, inst_tm = _gather_sc(t, h2, local_ids.reshape(-1),
                                 global_ids.reshape(-1))

    return _lstm(tok_tm, inst_tm, p)
